# baseline (speedup 1.0000x reference)
import functools

import jax
import jax.numpy as jnp
from jax import lax
from jax.experimental import pallas as pl
from jax.experimental.pallas import tpu as pltpu
from jax.experimental.pallas import tpu_sc as plsc

N_NODES = 10000
D_FEAT = 128
E_EDGES = 160000
P_STEPS = 12
Q_STEPS = 12

NC = 2
NS = 16
N_PAD = 10240
TILE_ROWS = N_PAD // NS
EDGES_PER_TILE = E_EDGES // NS
GROUP = 128
HALF = D_FEAT // 2
N_GROUPS = -(-EDGES_PER_TILE // GROUP) + (-(-EDGES_PER_TILE // GROUP)) % 2
KB = 2
NSB = N_GROUPS // KB
NG_ALLOC = N_GROUPS + KB
EPT_ALLOC = NG_ALLOC * GROUP

ROW_BLK = 2000

Z_PIECE = min(TILE_ROWS, GROUP)
Z_NUM = TILE_ROWS // Z_PIECE


def _make_spmm():
    mesh = plsc.VectorSubcoreMesh(
        core_axis_name="c", subcore_axis_name="s", num_cores=NC, num_subcores=NS
    )

    @functools.partial(
        pl.kernel,
        out_type=jax.ShapeDtypeStruct((2, N_PAD, HALF), jnp.float32),
        mesh=mesh,
        scratch_types=[
            pltpu.VMEM((NG_ALLOC, GROUP), jnp.int32),
            pltpu.VMEM((NG_ALLOC, GROUP), jnp.int32),
            pltpu.VMEM((EPT_ALLOC,), jnp.float32),
            pltpu.VMEM((KB * GROUP, HALF), jnp.float32),
            pltpu.VMEM((KB * GROUP, HALF), jnp.float32),
            pltpu.VMEM((KB * GROUP, HALF), jnp.float32),
            pltpu.VMEM_SHARED((N_PAD, HALF), jnp.float32),
            pltpu.SemaphoreType.DMA,
            pltpu.SemaphoreType.DMA,
            pltpu.SemaphoreType.DMA,
            pltpu.SemaphoreType.DMA,
            pltpu.SemaphoreType.DMA,
            pltpu.SemaphoreType.DMA,
        ],
        compiler_params=pltpu.CompilerParams(needs_layout_passes=False,
                                             use_tc_tiling_on_sc=False),
    )
    def spmm(x2_hbm, gi_hbm, si_hbm, w_hbm, out_hbm, gi_v, si_v, w_v,
             r0, r1, r2, acc, sg0, sg1, sg2, ss0, ss1, ss2):
        c = lax.axis_index("c")
        s = lax.axis_index("s")
        bufs = (r0, r1, r2)
        sg = (sg0, sg1, sg2)
        ss = (ss0, ss1, ss2)
        base = s * TILE_ROWS
        zero16 = jnp.zeros((16,), jnp.float32)

        def fire_gathers(sb, x):
            for k in range(KB):
                pltpu.async_copy(x2_hbm.at[gi_v.at[sb * KB + k]],
                                 bufs[x].at[pl.ds(k * GROUP, GROUP)], sg[x])

        def drain_gathers(x):
            for k in range(KB):
                pltpu.make_async_copy(x2_hbm.at[gi_v.at[0]],
                                      bufs[x].at[pl.ds(0, GROUP)],
                                      sg[x]).wait()

        def fire_scatters(sb, x):
            for k in range(KB):
                pltpu.async_copy(bufs[x].at[pl.ds(k * GROUP, GROUP)],
                                 acc.at[si_v.at[sb * KB + k]], ss[x],
                                 add=True)

        def drain_scatters(x):
            for k in range(KB):
                pltpu.make_async_copy(x2_hbm.at[gi_v.at[0]],
                                      bufs[x].at[pl.ds(0, GROUP)],
                                      ss[x]).wait()

        def scale(sb, x):
            buf = bufs[x]

            def sub16(g16, carry):
                w16 = w_v[pl.ds(sb * (KB * GROUP) + g16 * 16, 16)]
                e0 = g16 * 16
                for lane in range(16):
                    idx = jnp.full((16,), lane, jnp.int32)
                    wsp = w16.at[idx].get(mode="promise_in_bounds")
                    e = e0 + lane
                    for f in range(HALF // 16):
                        sl = pl.ds(f * 16, 16)
                        buf[e, sl] = buf[e, sl] * wsp
                return carry

            lax.fori_loop(0, KB * GROUP // 16, sub16, 0)

        def emit_phase():
            pltpu.sync_copy(gi_hbm.at[c, s], gi_v)
            pltpu.sync_copy(si_hbm.at[s], si_v)
            pltpu.sync_copy(w_hbm.at[s], w_v)

            def zrow(e, carry):
                for f in range(HALF // 16):
                    r0[e, pl.ds(f * 16, 16)] = zero16
                return carry

            lax.fori_loop(0, KB * GROUP, zrow, 0)
            off = 0
            while off < TILE_ROWS:
                sz = min(KB * GROUP, TILE_ROWS - off)
                pltpu.sync_copy(r0.at[pl.ds(0, sz)],
                                acc.at[pl.ds(base + off, sz)])
                off += sz
            plsc.subcore_barrier()

            def gstep(sb, x):
                nxt = (x + 1) % 3
                drain_scatters(nxt)
                fire_gathers(sb + 1, nxt)
                drain_gathers(x)
                scale(sb, x)
                fire_scatters(sb, x)

            fire_gathers(0, 0)
            fire_gathers(1, 1)
            drain_gathers(0)
            scale(0, 0)
            fire_scatters(0, 0)
            fire_gathers(2, 2)
            drain_gathers(1)
            scale(1, 1)
            fire_scatters(1, 1)

            def tri(m, carry):
                for k in range(3):
                    gstep(3 * m + 2 + k, (2 + k) % 3)
                return carry

            ntri = (NSB - 2) // 3
            lax.fori_loop(0, ntri, tri, 0)
            for sb in range(3 * ntri + 2, NSB):
                gstep(sb, sb % 3)
            drain_scatters((NSB - 2) % 3)
            drain_scatters((NSB - 1) % 3)
            drain_gathers(NSB % 3)
            plsc.subcore_barrier()

            for k5 in range(Z_NUM):
                pltpu.sync_copy(
                    acc.at[pl.ds(base + k5 * Z_PIECE, Z_PIECE)],
                    out_hbm.at[c, pl.ds(base + k5 * Z_PIECE, Z_PIECE)])

        emit_phase()

    return spmm


def _embx_tc(x2, W1, b1, W2, b2):
    M = x2.shape[0]

    def body(x_ref, w1_ref, b1_ref, w2_ref, b2_ref, o_ref):
        xb = jnp.broadcast_to(x_ref[...], (ROW_BLK, D_FEAT))
        wb = jnp.broadcast_to(w1_ref[...], (ROW_BLK, D_FEAT))
        t = jnp.maximum(xb * wb + b1_ref[...], 0.0)
        o_ref[...] = jnp.dot(t, w2_ref[...],
                             preferred_element_type=jnp.float32) + b2_ref[...]

    return pl.pallas_call(
        body,
        grid=(M // ROW_BLK,),
        in_specs=[
            pl.BlockSpec((ROW_BLK, 1), lambda i: (i, 0)),
            pl.BlockSpec((1, D_FEAT), lambda i: (0, 0)),
            pl.BlockSpec((1, D_FEAT), lambda i: (0, 0)),
            pl.BlockSpec((D_FEAT, D_FEAT), lambda i: (0, 0)),
            pl.BlockSpec((1, D_FEAT), lambda i: (0, 0)),
        ],
        out_specs=pl.BlockSpec((ROW_BLK, D_FEAT), lambda i: (i, 0)),
        out_shape=jax.ShapeDtypeStruct((M, D_FEAT), jnp.float32),
    )(x2, W1.reshape(1, D_FEAT), b1.reshape(1, D_FEAT), W2,
      b2.reshape(1, D_FEAT))


def _te_mlp_tc(onehot, W1, b1, W2, b2, mean, var):
    M, K = onehot.shape

    def body(oh_ref, w1_ref, b1_ref, w2_ref, b2_ref, m_ref, v_ref, o_ref):
        t = jnp.maximum(
            jnp.dot(oh_ref[...], w1_ref[...],
                    preferred_element_type=jnp.float32) + b1_ref[...], 0.0)
        t = jnp.dot(t, w2_ref[...],
                    preferred_element_type=jnp.float32) + b2_ref[...]
        o_ref[...] = (t - m_ref[...]) / jnp.sqrt(v_ref[...])

    return pl.pallas_call(
        body,
        out_shape=jax.ShapeDtypeStruct((M, D_FEAT), jnp.float32),
    )(onehot, W1, b1.reshape(1, D_FEAT), W2, b2.reshape(1, D_FEAT),
      mean.reshape(1, D_FEAT), var.reshape(1, D_FEAT))


def _precompute_P_tc(xt, sfx, sbx, Wx, bias):
    M = xt.shape[0]
    OW = Wx.shape[1]

    def body(x_ref, sf_ref, sb_ref, w_ref, b_ref, o_ref):
        cat = jnp.concatenate([x_ref[...], sf_ref[...], sb_ref[...]], axis=1)
        o_ref[...] = jnp.dot(cat, w_ref[...],
                             preferred_element_type=jnp.float32) + b_ref[...]

    return pl.pallas_call(
        body,
        grid=(M // ROW_BLK,),
        in_specs=[
            pl.BlockSpec((ROW_BLK, D_FEAT), lambda i: (i, 0)),
            pl.BlockSpec((ROW_BLK, D_FEAT), lambda i: (i, 0)),
            pl.BlockSpec((ROW_BLK, D_FEAT), lambda i: (i, 0)),
            pl.BlockSpec((3 * D_FEAT, OW), lambda i: (0, 0)),
            pl.BlockSpec((1, OW), lambda i: (0, 0)),
        ],
        out_specs=pl.BlockSpec((ROW_BLK, OW), lambda i: (i, 0)),
        out_shape=jax.ShapeDtypeStruct((M, OW), jnp.float32),
    )(xt, sfx, sbx, Wx, bias)


def _gates_tc(h, sfh, sbh, Pru, Wh_ru):
    def body(h_ref, sf_ref, sb_ref, p_ref, w_ref, rh_ref, u_ref):
        cat = jnp.concatenate([h_ref[...], sf_ref[...], sb_ref[...]], axis=1)
        g = p_ref[...] + jnp.dot(cat, w_ref[...],
                                 preferred_element_type=jnp.float32)
        ru = jax.nn.sigmoid(g)
        rh_ref[...] = ru[:, :D_FEAT] * h_ref[...]
        u_ref[...] = ru[:, D_FEAT:]

    return pl.pallas_call(
        body,
        grid=(N_NODES // ROW_BLK,),
        in_specs=[
            pl.BlockSpec((ROW_BLK, D_FEAT), lambda i: (i, 0)),
            pl.BlockSpec((ROW_BLK, D_FEAT), lambda i: (i, 0)),
            pl.BlockSpec((ROW_BLK, D_FEAT), lambda i: (i, 0)),
            pl.BlockSpec((ROW_BLK, 2 * D_FEAT), lambda i: (i, 0)),
            pl.BlockSpec((3 * D_FEAT, 2 * D_FEAT), lambda i: (0, 0)),
        ],
        out_specs=[
            pl.BlockSpec((ROW_BLK, D_FEAT), lambda i: (i, 0)),
            pl.BlockSpec((ROW_BLK, D_FEAT), lambda i: (i, 0)),
        ],
        out_shape=[
            jax.ShapeDtypeStruct((N_NODES, D_FEAT), jnp.float32),
            jax.ShapeDtypeStruct((N_NODES, D_FEAT), jnp.float32),
        ],
    )(h, sfh, sbh, Pru, Wh_ru)


def _update_tc(rh, sfr, sbr, Pc, u, h, Wh_c):
    def body(rh_ref, sf_ref, sb_ref, p_ref, u_ref, h_ref, w_ref, o_ref):
        cat = jnp.concatenate([rh_ref[...], sf_ref[...], sb_ref[...]], axis=1)
        cval = jnp.tanh(p_ref[...] + jnp.dot(
            cat, w_ref[...], preferred_element_type=jnp.float32))
        uv = u_ref[...]
        o_ref[...] = uv * h_ref[...] + (1.0 - uv) * cval

    return pl.pallas_call(
        body,
        grid=(N_NODES // ROW_BLK,),
        in_specs=[
            pl.BlockSpec((ROW_BLK, D_FEAT), lambda i: (i, 0)),
            pl.BlockSpec((ROW_BLK, D_FEAT), lambda i: (i, 0)),
            pl.BlockSpec((ROW_BLK, D_FEAT), lambda i: (i, 0)),
            pl.BlockSpec((ROW_BLK, D_FEAT), lambda i: (i, 0)),
            pl.BlockSpec((ROW_BLK, D_FEAT), lambda i: (i, 0)),
            pl.BlockSpec((ROW_BLK, D_FEAT), lambda i: (i, 0)),
            pl.BlockSpec((3 * D_FEAT, D_FEAT), lambda i: (0, 0)),
        ],
        out_specs=pl.BlockSpec((ROW_BLK, D_FEAT), lambda i: (i, 0)),
        out_shape=jax.ShapeDtypeStruct((N_NODES, D_FEAT), jnp.float32),
    )(rh, sfr, sbr, Pc, u, h, Wh_c)


def _head_tc(hmat, W1, b1, w2row, b2):
    M = hmat.shape[0]

    def body(h_ref, w1_ref, b1_ref, w2_ref, b2_ref, o_ref):
        t = jnp.maximum(
            jnp.dot(h_ref[...], w1_ref[...],
                    preferred_element_type=jnp.float32) + b1_ref[...], 0.0)
        o_ref[...] = jnp.sum(t * w2_ref[...], axis=1,
                             keepdims=True) + b2_ref[...]

    return pl.pallas_call(
        body,
        grid=(M // ROW_BLK,),
        in_specs=[
            pl.BlockSpec((ROW_BLK, D_FEAT), lambda i: (i, 0)),
            pl.BlockSpec((D_FEAT, D_FEAT), lambda i: (0, 0)),
            pl.BlockSpec((1, D_FEAT), lambda i: (0, 0)),
            pl.BlockSpec((1, D_FEAT), lambda i: (0, 0)),
            pl.BlockSpec((1, 1), lambda i: (0, 0)),
        ],
        out_specs=pl.BlockSpec((ROW_BLK, 1), lambda i: (i, 0)),
        out_shape=jax.ShapeDtypeStruct((M, 1), jnp.float32),
    )(hmat, W1, b1.reshape(1, D_FEAT), w2row, b2.reshape(1, 1))


def _split_w(W):
    D = D_FEAT
    Wx = jnp.concatenate([W[0:D], W[2 * D:3 * D], W[4 * D:5 * D]], axis=0)
    Wh = jnp.concatenate([W[D:2 * D], W[3 * D:4 * D], W[5 * D:6 * D]], axis=0)
    return Wx, Wh


def kernel(X, TE, edge_index, edge_w, te_W1, te_b1, te_W2, te_b2, te_mean,
           te_var, se_table, in_W1, in_b1, in_W2, in_b2, enc_Wru, enc_bru,
           enc_Wc, enc_bc, dec_Wru, dec_bru, dec_Wc, dec_bc, out_W1, out_b1,
           out_W2, out_b2):
    N, D = N_NODES, D_FEAT
    f32 = jnp.float32

    x2 = X.reshape(P_STEPS * N, 1)
    embX = _embx_tc(x2, in_W1, in_b1, in_W2, in_b2).reshape(P_STEPS, N, D)

    wd = jax.nn.one_hot(TE[0, :, 0], 7, dtype=f32)
    td = jax.nn.one_hot(TE[0, :, 1], 288, dtype=f32)
    onehot = jnp.concatenate([wd, td], axis=1)
    te = _te_mlp_tc(onehot, te_W1, te_b1, te_W2, te_b2, te_mean, te_var)

    se = se_table
    te_p = te[:P_STEPS]
    te_q = te[P_STEPS:]
    y_enc = embX + se[None]
    x_enc = y_enc + te_p[:, None, :]
    x_dec = jnp.broadcast_to(se[None] + te_q[:, None, :], (Q_STEPS, N, D))

    src_n = edge_index[0].astype(jnp.int32)
    dst_n = edge_index[1].astype(jnp.int32)
    out_deg = jnp.zeros((N,), f32).at[src_n].add(edge_w)
    in_deg = jnp.zeros((N,), f32).at[dst_n].add(edge_w)
    w_fwd = edge_w / jnp.maximum(out_deg[src_n], 1e-8)
    w_bwd = edge_w / jnp.maximum(in_deg[dst_n], 1e-8)
    rowsum_f = jnp.zeros((N,), f32).at[dst_n].add(w_fwd)
    rowsum_b = jnp.zeros((N,), f32).at[src_n].add(w_bwd)

    def prep(a, dtype):
        a = a.astype(dtype).reshape(NS, EDGES_PER_TILE)
        a = jnp.pad(a, ((0, 0), (0, EPT_ALLOC - EDGES_PER_TILE)))
        return a.reshape(NS, NG_ALLOC, GROUP)

    gp_f, gp_b = prep(src_n, jnp.int32), prep(dst_n, jnp.int32)
    gi_f = jnp.stack([2 * gp_f, 2 * gp_f + 1])
    gi_b = jnp.stack([2 * gp_b, 2 * gp_b + 1])
    si_f, si_b = prep(dst_n, jnp.int32), prep(src_n, jnp.int32)
    w_f = prep(w_fwd, f32).reshape(NS, EPT_ALLOC)
    w_b = prep(w_bwd, f32).reshape(NS, EPT_ALLOC)
    rowsum_f, rowsum_b, gi_f, gi_b = lax.optimization_barrier(
        (rowsum_f, rowsum_b, gi_f, gi_b))

    spmm = _make_spmm()

    def spmm_pair(x):
        x2 = x.reshape(2 * N, HALF)
        of = spmm(x2, gi_f, si_f, w_f)
        x2b, _ = lax.optimization_barrier((x2, of[0, 0, :1]))
        ob = spmm(x2b, gi_b, si_b, w_b)
        sf = jnp.concatenate([of[0, :N], of[1, :N]], axis=1)
        sb = jnp.concatenate([ob[0, :N], ob[1, :N]], axis=1)
        return sf, sb

    def pre_step(tok, y_t):
        y_t, tok = lax.optimization_barrier((y_t, tok))
        sf, sb = spmm_pair(y_t)
        return sf[0, :1], (sf, sb)

    ys = jnp.concatenate([y_enc, se[None]], axis=0)
    _, (sf_ys, sb_ys) = lax.scan(pre_step, jnp.zeros((1,), f32), ys)
    sf_yenc, sb_yenc = sf_ys[:P_STEPS], sb_ys[:P_STEPS]
    sf_se, sb_se = sf_ys[P_STEPS], sb_ys[P_STEPS]
    sfx_enc = sf_yenc + rowsum_f[None, :, None] * te_p[:, None, :]
    sbx_enc = sb_yenc + rowsum_b[None, :, None] * te_p[:, None, :]
    sfx_dec = sf_se[None] + rowsum_f[None, :, None] * te_q[:, None, :]
    sbx_dec = sb_se[None] + rowsum_b[None, :, None] * te_q[:, None, :]
    sfx = jnp.concatenate([sfx_enc, sfx_dec], axis=0)
    sbx = jnp.concatenate([sbx_enc, sbx_dec], axis=0)

    Wx_ru_e, Wh_ru_e = _split_w(enc_Wru)
    Wx_c_e, Wh_c_e = _split_w(enc_Wc)
    Wx_ru_d, Wh_ru_d = _split_w(dec_Wru)
    Wx_c_d, Wh_c_d = _split_w(dec_Wc)

    WxB_enc = jnp.concatenate([Wx_ru_e, Wx_c_e], axis=1)
    WxB_dec = jnp.concatenate([Wx_ru_d, Wx_c_d], axis=1)
    bB_enc = jnp.concatenate([enc_bru, enc_bc]).reshape(1, 3 * D)
    bB_dec = jnp.concatenate([dec_bru, dec_bc]).reshape(1, 3 * D)

    M_enc = P_STEPS * N
    P_enc = _precompute_P_tc(
        x_enc.reshape(M_enc, D), sfx[:P_STEPS].reshape(M_enc, D),
        sbx[:P_STEPS].reshape(M_enc, D), WxB_enc,
        bB_enc).reshape(P_STEPS, N, 3 * D)
    M_dec = Q_STEPS * N
    P_dec = _precompute_P_tc(
        x_dec.reshape(M_dec, D), sfx[P_STEPS:].reshape(M_dec, D),
        sbx[P_STEPS:].reshape(M_dec, D), WxB_dec,
        bB_dec).reshape(Q_STEPS, N, 3 * D)

    P_all = jnp.concatenate([P_enc, P_dec], axis=0)
    Wru_seq = jnp.concatenate(
        [jnp.broadcast_to(Wh_ru_e[None], (P_STEPS, 3 * D, 2 * D)),
         jnp.broadcast_to(Wh_ru_d[None], (Q_STEPS, 3 * D, 2 * D))], axis=0)
    Wc_seq = jnp.concatenate(
        [jnp.broadcast_to(Wh_c_e[None], (P_STEPS, 3 * D, D)),
         jnp.broadcast_to(Wh_c_d[None], (Q_STEPS, 3 * D, D))], axis=0)

    def step(h, xs):
        Pt, Wru, Wc = xs
        sfh, sbh = spmm_pair(h)
        rh, u = _gates_tc(h, sfh, sbh, Pt[:, :2 * D], Wru)
        sfr, sbr = spmm_pair(rh)
        h2 = _update_tc(rh, sfr, sbr, Pt[:, 2 * D:], u, h, Wc)
        return h2, h2

    _, hs = lax.scan(step, jnp.zeros((N, D), f32), (P_all, Wru_seq, Wc_seq))
    dec_hs = hs[P_STEPS:]

    y = _head_tc(dec_hs.reshape(Q_STEPS * N, D), out_W1, out_b1,
                 out_W2.reshape(1, D), out_b2)
    return y.reshape(1, Q_STEPS, N, 1)


# speedup vs baseline: 2.2186x; 2.2186x over previous
import functools

import jax
import jax.numpy as jnp
from jax import lax
from jax.experimental import pallas as pl
from jax.experimental.pallas import tpu as pltpu
from jax.experimental.pallas import tpu_sc as plsc

N_NODES = 10000
D_FEAT = 128
E_EDGES = 160000
P_STEPS = 12
Q_STEPS = 12

NC = 2
NS = 16
N_PAD = 10240
TILE_ROWS = N_PAD // NS
EDGES_PER_TILE = E_EDGES // NS
GROUP = 128
N_GROUPS = -(-EDGES_PER_TILE // GROUP)
EPT_PAD = N_GROUPS * GROUP

ROW_BLK = 2000

Z_PIECE = min(TILE_ROWS, GROUP)
Z_NUM = TILE_ROWS // Z_PIECE


def _make_spmm():
    mesh = plsc.VectorSubcoreMesh(
        core_axis_name="c", subcore_axis_name="s", num_cores=NC, num_subcores=NS
    )

    @functools.partial(
        pl.kernel,
        out_type=jax.ShapeDtypeStruct((2, N_PAD, D_FEAT), jnp.float32),
        mesh=mesh,
        scratch_types=[
            pltpu.VMEM((N_GROUPS, GROUP), jnp.int32),
            pltpu.VMEM((N_GROUPS, GROUP), jnp.int32),
            pltpu.VMEM((N_GROUPS, GROUP), jnp.float32),
            pltpu.VMEM((GROUP, D_FEAT), jnp.float32),
            pltpu.VMEM_SHARED((N_PAD, D_FEAT), jnp.float32),
            pltpu.SemaphoreType.DMA,
        ],
        compiler_params=pltpu.CompilerParams(needs_layout_passes=False),
    )
    def spmm(x_hbm, gi_hbm, si_hbm, w_hbm, out_hbm, gi_v, si_v, w_v, rows_v,
             acc, sem):
        c = lax.axis_index("c")
        s = lax.axis_index("s")
        pltpu.sync_copy(gi_hbm.at[c, s], gi_v)
        pltpu.sync_copy(si_hbm.at[c, s], si_v)
        pltpu.sync_copy(w_hbm.at[c, s], w_v)

        zero16 = jnp.zeros((16,), jnp.float32)

        def zrow(e, carry):
            for f in range(D_FEAT // 16):
                rows_v[e, pl.ds(f * 16, 16)] = zero16
            return carry

        lax.fori_loop(0, GROUP, zrow, 0)

        base = s * TILE_ROWS
        for k5 in range(Z_NUM):
            pltpu.sync_copy(rows_v.at[pl.ds(0, Z_PIECE)],
                            acc.at[pl.ds(base + k5 * Z_PIECE, Z_PIECE)])
        plsc.subcore_barrier()

        def chunk(j, carry):
            pltpu.async_copy(x_hbm.at[gi_v.at[j]], rows_v, sem).wait()

            def scale(e, c2):
                idx_j = jnp.full((16,), j, jnp.int32)
                idx_e = jnp.full((16,), e, jnp.int32)
                wsp = plsc.load_gather(w_v, [idx_j, idx_e])
                for f in range(D_FEAT // 16):
                    sl = pl.ds(f * 16, 16)
                    rows_v[e, sl] = rows_v[e, sl] * wsp
                return c2

            lax.fori_loop(0, GROUP, scale, 0)
            pltpu.sync_copy(rows_v, acc.at[si_v.at[j]], add=True)
            return carry

        lax.fori_loop(0, N_GROUPS, chunk, 0)
        plsc.subcore_barrier()

        for k5 in range(Z_NUM):
            pltpu.sync_copy(acc.at[pl.ds(base + k5 * Z_PIECE, Z_PIECE)],
                            out_hbm.at[c, pl.ds(base + k5 * Z_PIECE, Z_PIECE)])

    return spmm


def _embx_tc(x2, W1, b1, W2, b2):
    M = x2.shape[0]

    def body(x_ref, w1_ref, b1_ref, w2_ref, b2_ref, o_ref):
        xb = jnp.broadcast_to(x_ref[...], (ROW_BLK, D_FEAT))
        wb = jnp.broadcast_to(w1_ref[...], (ROW_BLK, D_FEAT))
        t = jnp.maximum(xb * wb + b1_ref[...], 0.0)
        o_ref[...] = jnp.dot(t, w2_ref[...],
                             preferred_element_type=jnp.float32) + b2_ref[...]

    return pl.pallas_call(
        body,
        grid=(M // ROW_BLK,),
        in_specs=[
            pl.BlockSpec((ROW_BLK, 1), lambda i: (i, 0)),
            pl.BlockSpec((1, D_FEAT), lambda i: (0, 0)),
            pl.BlockSpec((1, D_FEAT), lambda i: (0, 0)),
            pl.BlockSpec((D_FEAT, D_FEAT), lambda i: (0, 0)),
            pl.BlockSpec((1, D_FEAT), lambda i: (0, 0)),
        ],
        out_specs=pl.BlockSpec((ROW_BLK, D_FEAT), lambda i: (i, 0)),
        out_shape=jax.ShapeDtypeStruct((M, D_FEAT), jnp.float32),
    )(x2, W1.reshape(1, D_FEAT), b1.reshape(1, D_FEAT), W2,
      b2.reshape(1, D_FEAT))


def _te_mlp_tc(onehot, W1, b1, W2, b2, mean, var):
    M, K = onehot.shape

    def body(oh_ref, w1_ref, b1_ref, w2_ref, b2_ref, m_ref, v_ref, o_ref):
        t = jnp.maximum(
            jnp.dot(oh_ref[...], w1_ref[...],
                    preferred_element_type=jnp.float32) + b1_ref[...], 0.0)
        t = jnp.dot(t, w2_ref[...],
                    preferred_element_type=jnp.float32) + b2_ref[...]
        o_ref[...] = (t - m_ref[...]) / jnp.sqrt(v_ref[...])

    return pl.pallas_call(
        body,
        out_shape=jax.ShapeDtypeStruct((M, D_FEAT), jnp.float32),
    )(onehot, W1, b1.reshape(1, D_FEAT), W2, b2.reshape(1, D_FEAT),
      mean.reshape(1, D_FEAT), var.reshape(1, D_FEAT))


def _precompute_P_tc(xt, sfx, sbx, Wx, bias):
    M = xt.shape[0]
    OW = Wx.shape[1]

    def body(x_ref, sf_ref, sb_ref, w_ref, b_ref, o_ref):
        cat = jnp.concatenate([x_ref[...], sf_ref[...], sb_ref[...]], axis=1)
        o_ref[...] = jnp.dot(cat, w_ref[...],
                             preferred_element_type=jnp.float32) + b_ref[...]

    return pl.pallas_call(
        body,
        grid=(M // ROW_BLK,),
        in_specs=[
            pl.BlockSpec((ROW_BLK, D_FEAT), lambda i: (i, 0)),
            pl.BlockSpec((ROW_BLK, D_FEAT), lambda i: (i, 0)),
            pl.BlockSpec((ROW_BLK, D_FEAT), lambda i: (i, 0)),
            pl.BlockSpec((3 * D_FEAT, OW), lambda i: (0, 0)),
            pl.BlockSpec((1, OW), lambda i: (0, 0)),
        ],
        out_specs=pl.BlockSpec((ROW_BLK, OW), lambda i: (i, 0)),
        out_shape=jax.ShapeDtypeStruct((M, OW), jnp.float32),
    )(xt, sfx, sbx, Wx, bias)


def _gates_tc(h, sfh, sbh, Pru, Wh_ru):
    def body(h_ref, sf_ref, sb_ref, p_ref, w_ref, rh_ref, u_ref):
        cat = jnp.concatenate([h_ref[...], sf_ref[...], sb_ref[...]], axis=1)
        g = p_ref[...] + jnp.dot(cat, w_ref[...],
                                 preferred_element_type=jnp.float32)
        ru = jax.nn.sigmoid(g)
        rh_ref[...] = ru[:, :D_FEAT] * h_ref[...]
        u_ref[...] = ru[:, D_FEAT:]

    return pl.pallas_call(
        body,
        grid=(N_NODES // ROW_BLK,),
        in_specs=[
            pl.BlockSpec((ROW_BLK, D_FEAT), lambda i: (i, 0)),
            pl.BlockSpec((ROW_BLK, D_FEAT), lambda i: (i, 0)),
            pl.BlockSpec((ROW_BLK, D_FEAT), lambda i: (i, 0)),
            pl.BlockSpec((ROW_BLK, 2 * D_FEAT), lambda i: (i, 0)),
            pl.BlockSpec((3 * D_FEAT, 2 * D_FEAT), lambda i: (0, 0)),
        ],
        out_specs=[
            pl.BlockSpec((ROW_BLK, D_FEAT), lambda i: (i, 0)),
            pl.BlockSpec((ROW_BLK, D_FEAT), lambda i: (i, 0)),
        ],
        out_shape=[
            jax.ShapeDtypeStruct((N_NODES, D_FEAT), jnp.float32),
            jax.ShapeDtypeStruct((N_NODES, D_FEAT), jnp.float32),
        ],
    )(h, sfh, sbh, Pru, Wh_ru)


def _update_tc(rh, sfr, sbr, Pc, u, h, Wh_c):
    def body(rh_ref, sf_ref, sb_ref, p_ref, u_ref, h_ref, w_ref, o_ref):
        cat = jnp.concatenate([rh_ref[...], sf_ref[...], sb_ref[...]], axis=1)
        cval = jnp.tanh(p_ref[...] + jnp.dot(
            cat, w_ref[...], preferred_element_type=jnp.float32))
        uv = u_ref[...]
        o_ref[...] = uv * h_ref[...] + (1.0 - uv) * cval

    return pl.pallas_call(
        body,
        grid=(N_NODES // ROW_BLK,),
        in_specs=[
            pl.BlockSpec((ROW_BLK, D_FEAT), lambda i: (i, 0)),
            pl.BlockSpec((ROW_BLK, D_FEAT), lambda i: (i, 0)),
            pl.BlockSpec((ROW_BLK, D_FEAT), lambda i: (i, 0)),
            pl.BlockSpec((ROW_BLK, D_FEAT), lambda i: (i, 0)),
            pl.BlockSpec((ROW_BLK, D_FEAT), lambda i: (i, 0)),
            pl.BlockSpec((ROW_BLK, D_FEAT), lambda i: (i, 0)),
            pl.BlockSpec((3 * D_FEAT, D_FEAT), lambda i: (0, 0)),
        ],
        out_specs=pl.BlockSpec((ROW_BLK, D_FEAT), lambda i: (i, 0)),
        out_shape=jax.ShapeDtypeStruct((N_NODES, D_FEAT), jnp.float32),
    )(rh, sfr, sbr, Pc, u, h, Wh_c)


def _head_tc(hmat, W1, b1, w2row, b2):
    M = hmat.shape[0]

    def body(h_ref, w1_ref, b1_ref, w2_ref, b2_ref, o_ref):
        t = jnp.maximum(
            jnp.dot(h_ref[...], w1_ref[...],
                    preferred_element_type=jnp.float32) + b1_ref[...], 0.0)
        o_ref[...] = jnp.sum(t * w2_ref[...], axis=1,
                             keepdims=True) + b2_ref[...]

    return pl.pallas_call(
        body,
        grid=(M // ROW_BLK,),
        in_specs=[
            pl.BlockSpec((ROW_BLK, D_FEAT), lambda i: (i, 0)),
            pl.BlockSpec((D_FEAT, D_FEAT), lambda i: (0, 0)),
            pl.BlockSpec((1, D_FEAT), lambda i: (0, 0)),
            pl.BlockSpec((1, D_FEAT), lambda i: (0, 0)),
            pl.BlockSpec((1, 1), lambda i: (0, 0)),
        ],
        out_specs=pl.BlockSpec((ROW_BLK, 1), lambda i: (i, 0)),
        out_shape=jax.ShapeDtypeStruct((M, 1), jnp.float32),
    )(hmat, W1, b1.reshape(1, D_FEAT), w2row, b2.reshape(1, 1))


def _split_w(W):
    D = D_FEAT
    Wx = jnp.concatenate([W[0:D], W[2 * D:3 * D], W[4 * D:5 * D]], axis=0)
    Wh = jnp.concatenate([W[D:2 * D], W[3 * D:4 * D], W[5 * D:6 * D]], axis=0)
    return Wx, Wh


def kernel(X, TE, edge_index, edge_w, te_W1, te_b1, te_W2, te_b2, te_mean,
           te_var, se_table, in_W1, in_b1, in_W2, in_b2, enc_Wru, enc_bru,
           enc_Wc, enc_bc, dec_Wru, dec_bru, dec_Wc, dec_bc, out_W1, out_b1,
           out_W2, out_b2):
    N, D = N_NODES, D_FEAT
    f32 = jnp.float32

    x2 = X.reshape(P_STEPS * N, 1)
    embX = _embx_tc(x2, in_W1, in_b1, in_W2, in_b2).reshape(P_STEPS, N, D)

    wd = jax.nn.one_hot(TE[0, :, 0], 7, dtype=f32)
    td = jax.nn.one_hot(TE[0, :, 1], 288, dtype=f32)
    onehot = jnp.concatenate([wd, td], axis=1)
    te = _te_mlp_tc(onehot, te_W1, te_b1, te_W2, te_b2, te_mean, te_var)

    se = se_table
    te_p = te[:P_STEPS]
    te_q = te[P_STEPS:]
    y_enc = embX + se[None]
    x_enc = y_enc + te_p[:, None, :]
    x_dec = jnp.broadcast_to(se[None] + te_q[:, None, :], (Q_STEPS, N, D))

    src_n = edge_index[0].astype(jnp.int32)
    dst_n = edge_index[1].astype(jnp.int32)
    out_deg = jnp.zeros((N,), f32).at[src_n].add(edge_w)
    in_deg = jnp.zeros((N,), f32).at[dst_n].add(edge_w)
    w_fwd = edge_w / jnp.maximum(out_deg[src_n], 1e-8)
    w_bwd = edge_w / jnp.maximum(in_deg[dst_n], 1e-8)

    def prep(a, dtype):
        a = a.astype(dtype).reshape(2, NS, EDGES_PER_TILE)
        a = jnp.pad(a, ((0, 0), (0, 0), (0, EPT_PAD - EDGES_PER_TILE)))
        return a.reshape(2, NS, N_GROUPS, GROUP)

    gi = prep(jnp.stack([src_n, dst_n]), jnp.int32)
    si = prep(jnp.stack([dst_n, src_n]), jnp.int32)
    wv = prep(jnp.stack([w_fwd, w_bwd]), f32)

    spmm = _make_spmm()

    def spmm_pair(x):
        o = spmm(x, gi, si, wv)
        return o[0, :N], o[1, :N]

    def pre_step(carry, y_t):
        sf, sb = spmm_pair(y_t)
        return carry, (sf, sb)

    ys = jnp.concatenate(
        [y_enc, se[None], jnp.ones((1, N, D), f32)], axis=0)
    _, (sf_ys, sb_ys) = lax.scan(pre_step, 0, ys)
    sf_yenc, sb_yenc = sf_ys[:P_STEPS], sb_ys[:P_STEPS]
    sf_se, sb_se = sf_ys[P_STEPS], sb_ys[P_STEPS]
    rowsum_f = sf_ys[P_STEPS + 1][:, :1]
    rowsum_b = sb_ys[P_STEPS + 1][:, :1]

    sfx_enc = sf_yenc + rowsum_f[None] * te_p[:, None, :]
    sbx_enc = sb_yenc + rowsum_b[None] * te_p[:, None, :]
    sfx_dec = sf_se[None] + rowsum_f[None] * te_q[:, None, :]
    sbx_dec = sb_se[None] + rowsum_b[None] * te_q[:, None, :]

    Wx_ru_e, Wh_ru_e = _split_w(enc_Wru)
    Wx_c_e, Wh_c_e = _split_w(enc_Wc)
    Wx_ru_d, Wh_ru_d = _split_w(dec_Wru)
    Wx_c_d, Wh_c_d = _split_w(dec_Wc)

    WxB_enc = jnp.concatenate([Wx_ru_e, Wx_c_e], axis=1)
    WxB_dec = jnp.concatenate([Wx_ru_d, Wx_c_d], axis=1)
    bB_enc = jnp.concatenate([enc_bru, enc_bc]).reshape(1, 3 * D)
    bB_dec = jnp.concatenate([dec_bru, dec_bc]).reshape(1, 3 * D)

    M_enc = P_STEPS * N
    P_enc = _precompute_P_tc(
        x_enc.reshape(M_enc, D), sfx_enc.reshape(M_enc, D),
        sbx_enc.reshape(M_enc, D), WxB_enc,
        bB_enc).reshape(P_STEPS, N, 3 * D)
    M_dec = Q_STEPS * N
    P_dec = _precompute_P_tc(
        x_dec.reshape(M_dec, D), sfx_dec.reshape(M_dec, D),
        sbx_dec.reshape(M_dec, D), WxB_dec,
        bB_dec).reshape(Q_STEPS, N, 3 * D)

    def make_step(Wh_ru, Wh_c):
        def step(h, Pt):
            sfh, sbh = spmm_pair(h)
            rh, u = _gates_tc(h, sfh, sbh, Pt[:, :2 * D], Wh_ru)
            sfr, sbr = spmm_pair(rh)
            h2 = _update_tc(rh, sfr, sbr, Pt[:, 2 * D:], u, h, Wh_c)
            return h2, h2
        return step

    h0 = jnp.zeros((N, D), f32)
    hT, _ = lax.scan(make_step(Wh_ru_e, Wh_c_e), h0, P_enc)
    _, dec_hs = lax.scan(make_step(Wh_ru_d, Wh_c_d), hT, P_dec)

    y = _head_tc(dec_hs.reshape(Q_STEPS * N, D), out_W1, out_b1,
                 out_W2.reshape(1, D), out_b2)
    return y.reshape(1, Q_STEPS, N, 1)
